# baseline (speedup 1.0000x reference)
import functools

import jax
import jax.numpy as jnp
from jax import lax
from jax.experimental import pallas as pl
from jax.experimental.pallas import tpu as pltpu
from jax.experimental.pallas import tpu_sc as plsc

BATCH = 16384
HIDDEN = 64
CHUNK = 128


@functools.cache
def _make_kernel(B, D):
    info = plsc.get_sparse_core_info()
    NC, NS = info.num_cores, info.num_subcores
    NW = NC * NS
    b_per_w = B // NW
    n_chunks = b_per_w // CHUNK
    mesh = plsc.VectorSubcoreMesh(core_axis_name="c", subcore_axis_name="s")

    @functools.partial(
        pl.kernel,
        mesh=mesh,
        out_type=jax.ShapeDtypeStruct((B, D), jnp.float32),
        compiler_params=pltpu.CompilerParams(use_tc_tiling_on_sc=False),
        scratch_types=[
            pltpu.VMEM((b_per_w,), jnp.int32),
            pltpu.VMEM((b_per_w, D), jnp.float32),
            pltpu.SemaphoreType.DMA,
        ],
    )
    def k(labels_hbm, table_hbm, out_hbm, idx_v, rows_v, sem):
        wid = lax.axis_index("s") * NC + lax.axis_index("c")
        base = wid * b_per_w
        pltpu.sync_copy(labels_hbm.at[pl.ds(base, b_per_w)], idx_v)
        copies = [
            pltpu.async_copy(
                table_hbm.at[idx_v.at[pl.ds(j * CHUNK, CHUNK)]],
                rows_v.at[pl.ds(j * CHUNK, CHUNK)],
                sem,
            )
            for j in range(n_chunks)
        ]
        for c in copies:
            c.wait()
        pltpu.sync_copy(rows_v, out_hbm.at[pl.ds(base, b_per_w)])

    return k


def kernel(labels, train, table):
    k = _make_kernel(BATCH, HIDDEN)
    return k(labels.astype(jnp.int32), table)


# speedup vs baseline: 3.2034x; 3.2034x over previous
import functools

import jax
import jax.numpy as jnp
from jax import lax
from jax.experimental import pallas as pl
from jax.experimental.pallas import tpu as pltpu
from jax.experimental.pallas import tpu_sc as plsc

BATCH = 16384
HIDDEN = 64
NBUF = 8


@functools.cache
def _make_kernel(B, D):
    info = plsc.get_sparse_core_info()
    NC, NS, L = info.num_cores, info.num_subcores, info.num_lanes
    NW = NC * NS
    b_per_w = B // NW
    mesh = plsc.VectorSubcoreMesh(core_axis_name="c", subcore_axis_name="s")

    @functools.partial(
        pl.kernel,
        mesh=mesh,
        out_type=jax.ShapeDtypeStruct((D, B), jnp.float32),
        compiler_params=pltpu.CompilerParams(
            use_tc_tiling_on_sc=True, needs_layout_passes=False),
        scratch_types=[
            pltpu.VMEM((b_per_w + 16,), jnp.int32),
            pltpu.VMEM((NBUF, D, 128), jnp.float32),
            pltpu.VMEM((b_per_w // 128, D, 128), jnp.float32),
            pltpu.SemaphoreType.DMA,
        ],
    )
    def k(labels_hbm, tT_hbm, outT_hbm, labels_s, blocks_v, outbuf_v, sem):
        wid = lax.axis_index("s") * NC + lax.axis_index("c")
        base = wid * b_per_w
        pltpu.sync_copy(labels_hbm.at[pl.ds(base, b_per_w)],
                        labels_s.at[pl.ds(0, b_per_w)])

        def lab_at(j):
            return labels_s[pl.ds(j, 16)][0]

        lane = lax.iota(jnp.int32, L)

        def descriptor(j, u):
            lab = lab_at(j)
            off = pl.multiple_of((lab >> 7) * 128, 128)
            return pltpu.make_async_copy(
                tT_hbm.at[:, pl.ds(off, 128)],
                blocks_v.at[u], sem)

        def select(j, u):
            lab = lab_at(j)
            l_vec = jnp.full((L,), lab & 127, jnp.int32)
            g_vec = jnp.full((L,), j >> 7, jnp.int32)
            j_vec = jnp.full((L,), j & 127, jnp.int32)
            blk = blocks_v.at[u]
            for cg in range(D // L):
                c_vec = lane + (cg * L)
                v = plsc.load_gather(blk, [c_vec, l_vec])
                plsc.store_scatter(outbuf_v, [g_vec, c_vec, j_vec], v)

        for u in range(NBUF):
            descriptor(u, u).start()

        def step_body(step, _):
            for u in range(NBUF):
                j = step * NBUF + u
                descriptor(j, u).wait()
                select(j, u)
                descriptor(j + NBUF, u).start()
            return 0

        n_steps = b_per_w // NBUF
        lax.fori_loop(0, n_steps - 1, step_body, 0)
        for u in range(NBUF):
            j = (n_steps - 1) * NBUF + u
            descriptor(j, u).wait()
            select(j, u)

        for g in range(b_per_w // 128):
            pltpu.sync_copy(outbuf_v.at[g],
                            outT_hbm.at[:, pl.ds(base + g * 128, 128)])

    return k


def kernel(labels, train, table):
    k = _make_kernel(BATCH, HIDDEN)
    outT = k(labels.astype(jnp.int32), table.T)
    return outT.T


# speedup vs baseline: 3.2094x; 1.0019x over previous
import functools

import jax
import jax.numpy as jnp
from jax import lax
from jax.experimental import pallas as pl
from jax.experimental.pallas import tpu as pltpu
from jax.experimental.pallas import tpu_sc as plsc

BATCH = 16384
HIDDEN = 64
NBUF = 8


@functools.cache
def _make_kernel(B, D):
    info = plsc.get_sparse_core_info()
    NC, NS, L = info.num_cores, info.num_subcores, info.num_lanes
    NW = NC * NS
    b_per_w = B // NW
    mesh = plsc.VectorSubcoreMesh(core_axis_name="c", subcore_axis_name="s")

    @functools.partial(
        pl.kernel,
        mesh=mesh,
        out_type=jax.ShapeDtypeStruct((D, B), jnp.float32),
        compiler_params=pltpu.CompilerParams(
            use_tc_tiling_on_sc=True, needs_layout_passes=False),
        scratch_types=[
            pltpu.VMEM((b_per_w + 16,), jnp.int32),
            pltpu.VMEM((NBUF, D, 128), jnp.float32),
            pltpu.VMEM((b_per_w // 128, D, 128), jnp.float32),
            pltpu.SemaphoreType.DMA,
        ],
    )
    def k(labels_hbm, tT_hbm, outT_hbm, labels_s, blocks_v, outbuf_v, sem):
        wid = lax.axis_index("s") * NC + lax.axis_index("c")
        base = wid * b_per_w
        pltpu.sync_copy(labels_hbm.at[pl.ds(base, b_per_w)],
                        labels_s.at[pl.ds(0, b_per_w)])

        def lab_at(j):
            return labels_s[pl.ds(j, 16)][0]

        lane = lax.iota(jnp.int32, L)

        def descriptor(j, u):
            lab = lab_at(j)
            off = pl.multiple_of((lab >> 7) * 128, 128)
            return pltpu.make_async_copy(
                tT_hbm.at[:, pl.ds(off, 128)],
                blocks_v.at[u], sem)

        def select(j, u):
            lab = lab_at(j)
            l_vec = jnp.full((L,), lab & 127, jnp.int32)
            g_vec = jnp.full((L,), j >> 7, jnp.int32)
            j_vec = jnp.full((L,), j & 127, jnp.int32)
            blk = blocks_v.at[u]
            for cg in range(D // L):
                c_vec = lane + (cg * L)
                v = plsc.load_gather(blk, [c_vec, l_vec])
                plsc.store_scatter(outbuf_v, [g_vec, c_vec, j_vec], v)

        for u in range(NBUF):
            descriptor(u, u).start()

        def step_body(step, _):
            for u in range(NBUF):
                j = step * NBUF + u
                descriptor(j, u).wait()
                select(j, u)
                descriptor(j + NBUF, u).start()
            return 0

        n_steps = b_per_w // NBUF
        lax.fori_loop(0, n_steps - 1, step_body, 0)
        for u in range(NBUF):
            j = (n_steps - 1) * NBUF + u
            descriptor(j, u).wait()
            select(j, u)

        outs = [
            pltpu.async_copy(outbuf_v.at[g],
                             outT_hbm.at[:, pl.ds(base + g * 128, 128)], sem)
            for g in range(b_per_w // 128)
        ]
        for o in outs:
            o.wait()

    return k


def kernel(labels, train, table):
    k = _make_kernel(BATCH, HIDDEN)
    outT = k(labels.astype(jnp.int32), table.T)
    return outT.T
